# baseline (speedup 1.0000x reference)
import functools

import jax
import jax.numpy as jnp
from jax import lax
from jax.experimental import pallas as pl
from jax.experimental.pallas import tpu as pltpu
from jax.experimental.pallas import tpu_sc as plsc

NUM_CODES = 1024
DIM = 64
BATCH = 32
SEQ = 576
TOK = BATCH * SEQ
ROWS = 4
BLK = ROWS * SEQ
NBLK = BATCH // ROWS


def _vq_body(x_hbm, cb_ref, idx_ref, vq_ref, perp_ref,
             hist_ref, b_ref, iotaf_ref, cb2_ref, xbuf, sems, acc_ref):
    k = pl.program_id(0)

    def _x_copy(step, slot):
        return pltpu.make_async_copy(
            x_hbm.at[pl.ds(step * ROWS, ROWS)], xbuf.at[slot], sems.at[slot])

    @pl.when(k == 0)
    def _init():
        hist_ref[...] = jnp.zeros_like(hist_ref)
        cb = cb_ref[...]
        b_ref[...] = jnp.sum(cb * cb, axis=1)[None, :]
        cb2_ref[...] = -2.0 * cb
        iotaf_ref[...] = lax.broadcasted_iota(
            jnp.int32, (1, NUM_CODES), 1).astype(jnp.float32)
        acc_ref[0, 0] = 0.0
        _x_copy(0, 0).start()

    @pl.when(k + 1 < NBLK)
    def _prefetch():
        _x_copy(k + 1, (k + 1) % 2).start()

    _x_copy(k, k % 2).wait()
    x = xbuf[k % 2].reshape(BLK, DIM)
    s2 = lax.dot_general(x, cb2_ref[...], (((1,), (1,)), ((), ())),
                         preferred_element_type=jnp.float32)
    a = jnp.sum(x * x, axis=1, keepdims=True)
    dist = (a + b_ref[...]) + s2

    minval = jnp.min(dist, axis=1, keepdims=True)
    iota = jnp.broadcast_to(iotaf_ref[...], dist.shape)
    idxf = jnp.min(jnp.where(dist == minval, iota, 65536.0), axis=1)
    idx_ref[pl.ds(k * BLK, BLK)] = idxf.astype(jnp.int32)

    onehot = (iota == idxf[:, None]).astype(jnp.float32)
    ones_row = jnp.ones((1, BLK), jnp.float32)
    hist_ref[...] += jnp.dot(ones_row, onehot,
                             preferred_element_type=jnp.float32)
    acc_ref[0, 0] += jnp.sum(minval)

    @pl.when(k == NBLK - 1)
    def _fini():
        p = hist_ref[0, :] * (1.0 / TOK)
        ent = jnp.sum(p * jnp.log(p + 1e-10))
        perp_ref[...] = jnp.full((1, 1), jnp.exp(-ent), jnp.float32)
        v = acc_ref[0, 0] * (1.0 / (TOK * DIM))
        vq_ref[...] = jnp.full((1, 1), v + 0.25 * v, jnp.float32)


_vq_call = pl.pallas_call(
    _vq_body,
    grid=(NBLK,),
    in_specs=[
        pl.BlockSpec(memory_space=pl.ANY),
        pl.BlockSpec((NUM_CODES, DIM), lambda k: (0, 0)),
    ],
    out_specs=[
        pl.BlockSpec((TOK,), lambda k: (0,)),
        pl.BlockSpec((1, 1), lambda k: (0, 0)),
        pl.BlockSpec((1, 1), lambda k: (0, 0)),
    ],
    out_shape=[
        jax.ShapeDtypeStruct((TOK,), jnp.int32),
        jax.ShapeDtypeStruct((1, 1), jnp.float32),
        jax.ShapeDtypeStruct((1, 1), jnp.float32),
    ],
    scratch_shapes=[
        pltpu.VMEM((1, NUM_CODES), jnp.float32),
        pltpu.VMEM((1, NUM_CODES), jnp.float32),
        pltpu.VMEM((1, NUM_CODES), jnp.float32),
        pltpu.VMEM((NUM_CODES, DIM), jnp.float32),
        pltpu.VMEM((2, ROWS, SEQ, DIM), jnp.float32),
        pltpu.SemaphoreType.DMA((2,)),
        pltpu.SMEM((1, 1), jnp.float32),
    ],
)


_NC = 2
_NS = 16
NW = _NC * _NS
TPW = TOK // NW
CH = 96
NCH = TPW // CH


@functools.cache
def _sc_gather():
    mesh = plsc.VectorSubcoreMesh(core_axis_name="c", subcore_axis_name="s")

    @functools.partial(
        pl.kernel,
        mesh=mesh,
        compiler_params=pltpu.CompilerParams(use_tc_tiling_on_sc=False),
        out_type=jax.ShapeDtypeStruct((BATCH, SEQ, DIM), jnp.float32),
        scratch_types=[
            pltpu.VMEM((TPW,), jnp.int32),
            pltpu.VMEM((TPW, DIM), jnp.float32),
            pltpu.SemaphoreType.DMA,
        ],
    )
    def gather(cb_hbm, idx_hbm, out_hbm, idx_v, rows_v, sem):
        w = lax.axis_index("s") * _NC + lax.axis_index("c")
        pltpu.sync_copy(idx_hbm.at[pl.ds(w * TPW, TPW)], idx_v)
        for j in range(NCH):
            pltpu.async_copy(cb_hbm.at[idx_v.at[pl.ds(j * CH, CH)]],
                             rows_v.at[pl.ds(j * CH, CH)], sem).wait()
        pltpu.sync_copy(rows_v, out_hbm.at[w])

    return gather


def kernel(inputs, codebook):
    idx_flat, vq, perp = _vq_call(inputs, codebook)
    quantized = _sc_gather()(codebook, idx_flat)
    return (
        quantized,
        idx_flat.reshape(BATCH, SEQ),
        vq[0, 0],
        perp[0, 0],
    )


# speedup vs baseline: 1.0108x; 1.0108x over previous
import functools

import jax
import jax.numpy as jnp
from jax import lax
from jax.experimental import pallas as pl
from jax.experimental.pallas import tpu as pltpu
from jax.experimental.pallas import tpu_sc as plsc

NUM_CODES = 1024
DIM = 64
BATCH = 32
SEQ = 576
TOK = BATCH * SEQ
ROWS = 4
BLK = ROWS * SEQ
NBLK = BATCH // ROWS


def _vq_body(x_ref, cb_ref, idx_ref, vq_ref, perp_ref,
             hist_ref, b_ref, iotaf_ref, cb2_ref, acc_ref):
    k = pl.program_id(0)

    @pl.when(k == 0)
    def _init():
        hist_ref[...] = jnp.zeros_like(hist_ref)
        cb = cb_ref[...]
        b_ref[...] = jnp.sum(cb * cb, axis=1)[None, :]
        cb2_ref[...] = -2.0 * cb
        iotaf_ref[...] = lax.broadcasted_iota(
            jnp.int32, (1, NUM_CODES), 1).astype(jnp.float32)
        acc_ref[0, 0] = 0.0

    x = x_ref[...].reshape(BLK, DIM)
    s2 = lax.dot_general(x, cb2_ref[...], (((1,), (1,)), ((), ())),
                         preferred_element_type=jnp.float32)
    a = jnp.sum(x * x, axis=1, keepdims=True)
    dist = (a + b_ref[...]) + s2

    minval = jnp.min(dist, axis=1, keepdims=True)
    iota = jnp.broadcast_to(iotaf_ref[...], dist.shape)
    idxf = jnp.min(jnp.where(dist == minval, iota, 65536.0), axis=1)
    idx_ref[pl.ds(k * BLK, BLK)] = idxf.astype(jnp.int32)

    onehot = (iota == idxf[:, None]).astype(jnp.float32)
    ones_row = jnp.ones((1, BLK), jnp.float32)
    hist_ref[...] += jnp.dot(ones_row, onehot,
                             preferred_element_type=jnp.float32)
    acc_ref[0, 0] += jnp.sum(minval)

    @pl.when(k == NBLK - 1)
    def _fini():
        p = hist_ref[0, :] * (1.0 / TOK)
        ent = jnp.sum(p * jnp.log(p + 1e-10))
        perp_ref[...] = jnp.full((1, 1), jnp.exp(-ent), jnp.float32)
        v = acc_ref[0, 0] * (1.0 / (TOK * DIM))
        vq_ref[...] = jnp.full((1, 1), v + 0.25 * v, jnp.float32)


_vq_call = pl.pallas_call(
    _vq_body,
    grid=(NBLK,),
    in_specs=[
        pl.BlockSpec((ROWS, SEQ, DIM), lambda k: (k, 0, 0)),
        pl.BlockSpec((NUM_CODES, DIM), lambda k: (0, 0)),
    ],
    out_specs=[
        pl.BlockSpec((TOK,), lambda k: (0,)),
        pl.BlockSpec((1, 1), lambda k: (0, 0)),
        pl.BlockSpec((1, 1), lambda k: (0, 0)),
    ],
    out_shape=[
        jax.ShapeDtypeStruct((TOK,), jnp.int32),
        jax.ShapeDtypeStruct((1, 1), jnp.float32),
        jax.ShapeDtypeStruct((1, 1), jnp.float32),
    ],
    scratch_shapes=[
        pltpu.VMEM((1, NUM_CODES), jnp.float32),
        pltpu.VMEM((1, NUM_CODES), jnp.float32),
        pltpu.VMEM((1, NUM_CODES), jnp.float32),
        pltpu.VMEM((NUM_CODES, DIM), jnp.float32),
        pltpu.SMEM((1, 1), jnp.float32),
    ],
)


_NC = 2
_NS = 16
NW = _NC * _NS
TPW = TOK // NW
CH = 96
NCH = TPW // CH


@functools.cache
def _sc_gather():
    mesh = plsc.VectorSubcoreMesh(core_axis_name="c", subcore_axis_name="s")

    @functools.partial(
        pl.kernel,
        mesh=mesh,
        compiler_params=pltpu.CompilerParams(use_tc_tiling_on_sc=False),
        out_type=jax.ShapeDtypeStruct((BATCH, SEQ, DIM), jnp.float32),
        scratch_types=[
            pltpu.VMEM((TPW,), jnp.int32),
            pltpu.VMEM((TPW, DIM), jnp.float32),
            pltpu.SemaphoreType.DMA,
        ],
    )
    def gather(cb_hbm, idx_hbm, out_hbm, idx_v, rows_v, sem):
        w = lax.axis_index("s") * _NC + lax.axis_index("c")
        pltpu.sync_copy(idx_hbm.at[pl.ds(w * TPW, TPW)], idx_v)
        copies = [
            pltpu.async_copy(cb_hbm.at[idx_v.at[pl.ds(j * CH, CH)]],
                             rows_v.at[pl.ds(j * CH, CH)], sem)
            for j in range(NCH)
        ]
        for cp in copies:
            cp.wait()
        pltpu.sync_copy(rows_v, out_hbm.at[w])

    return gather


def kernel(inputs, codebook):
    idx_flat, vq, perp = _vq_call(inputs, codebook)
    quantized = _sc_gather()(codebook, idx_flat)
    return (
        quantized,
        idx_flat.reshape(BATCH, SEQ),
        vq[0, 0],
        perp[0, 0],
    )
